# baseline (speedup 1.0000x reference)
import functools

import jax
import jax.numpy as jnp
from jax import lax
from jax.experimental import pallas as pl
from jax.experimental.pallas import tpu as pltpu
from jax.experimental.pallas import tpu_sc as plsc

_TOTAL = 16384
_D = 512
_NB = 16
_BLK = 4096
_NW = 16
_CHUNK = _TOTAL // _NW
_NV = _CHUNK // 16


def _tc_body(x_ref, w_ref, benc_ref, waggt_ref, r_ref):
    h = jnp.maximum(
        jnp.dot(x_ref[...], w_ref[...], preferred_element_type=jnp.float32)
        + benc_ref[...], 0.0)
    r_ref[...] = jnp.sum(h * waggt_ref[...], axis=1, keepdims=True)


def _sc_body(r_hbm, starts_hbm, ends_hbm, bagg_hbm, out_hbm,
             r_v, st_v, en_v, bg_v, acc_v, all_v, out_v, shared, sem):
    wid = lax.axis_index("s")
    base = wid * _CHUNK
    pltpu.sync_copy(r_hbm.at[pl.ds(base, _CHUNK)], r_v)
    pltpu.sync_copy(starts_hbm, st_v)
    pltpu.sync_copy(ends_hbm, en_v)

    lane = lax.iota(jnp.int32, 16)
    starts_vec = st_v[...]
    ends_vec = en_v[...]
    out = jnp.zeros((16,), jnp.float32)
    for b in range(_NB):
        sb = starts_vec[b]
        eb = ends_vec[b]
        lo_r = jnp.clip(sb - base, 0, _CHUNK)
        hi_r = jnp.clip(eb - base, 0, _CHUNK)
        lo_v = lo_r // 16
        hi_v = jnp.maximum((hi_r + 15) // 16, lo_v)

        def body(v, a, sb=sb, eb=eb):
            rv = r_v[pl.ds(v * 16, 16)]
            g = base + v * 16 + lane
            return a + jnp.where((g >= sb) & (g < eb), rv, 0.0)

        acc_b = lax.fori_loop(lo_v, hi_v, body,
                              jnp.zeros((16,), jnp.float32))
        tot_b = acc_b[0]
        for l in range(1, 16):
            tot_b = tot_b + acc_b[l]
        out = jnp.where(lane == b, tot_b, out)
    acc_v[...] = out

    pltpu.sync_copy(acc_v, shared.at[pl.ds(wid * 16, 16)])
    plsc.subcore_barrier()

    @pl.when(wid == 0)
    def _fin():
        pltpu.sync_copy(shared, all_v)
        pltpu.sync_copy(bagg_hbm, bg_v)
        tot = jnp.zeros((16,), jnp.float32)
        for w in range(_NW):
            tot = tot + all_v[pl.ds(w * 16, 16)]
        counts = jnp.maximum((en_v[...] - st_v[...]).astype(jnp.float32), 1.0)
        out_v[...] = tot / counts + bg_v[...]
        pltpu.sync_copy(out_v, out_hbm)


def kernel(x, bag_sizes, W_enc, b_enc, W_agg, b_agg):
    starts = bag_sizes[:_NB]
    ends = bag_sizes[1:]
    waggt = W_agg.reshape(1, _D)
    benc = b_enc.reshape(1, _D)
    bagg16 = jnp.broadcast_to(b_agg, (_NB,))

    grid = _TOTAL // _BLK
    r = pl.pallas_call(
        _tc_body,
        grid=(grid,),
        in_specs=[
            pl.BlockSpec((_BLK, _D), lambda i: (i, 0)),
            pl.BlockSpec((_D, _D), lambda i: (0, 0)),
            pl.BlockSpec((1, _D), lambda i: (0, 0)),
            pl.BlockSpec((1, _D), lambda i: (0, 0)),
        ],
        out_specs=pl.BlockSpec((_BLK, 1), lambda i: (i, 0)),
        out_shape=jax.ShapeDtypeStruct((_TOTAL, 1), jnp.float32),
        compiler_params=pltpu.CompilerParams(
            dimension_semantics=("arbitrary",)),
    )(x, W_enc, benc, waggt)

    mesh = plsc.VectorSubcoreMesh(core_axis_name="c", subcore_axis_name="s",
                                  num_cores=1)
    f = pl.kernel(
        _sc_body,
        out_type=jax.ShapeDtypeStruct((_NB,), jnp.float32),
        mesh=mesh,
        scratch_types=[
            pltpu.VMEM((_CHUNK,), jnp.float32),
            pltpu.VMEM((16,), jnp.int32),
            pltpu.VMEM((16,), jnp.int32),
            pltpu.VMEM((16,), jnp.float32),
            pltpu.VMEM((16,), jnp.float32),
            pltpu.VMEM((_NW * 16,), jnp.float32),
            pltpu.VMEM((16,), jnp.float32),
            pltpu.VMEM_SHARED((_NW * 16,), jnp.float32),
            pltpu.SemaphoreType.DMA,
        ],
    )
    out = f(r.reshape(_TOTAL), starts, ends, bagg16)
    return out.reshape(_NB, 1)


# speedup vs baseline: 2.1038x; 2.1038x over previous
import jax
import jax.numpy as jnp
from jax.experimental import pallas as pl
from jax.experimental.pallas import tpu as pltpu

_TOTAL = 16384
_D = 512
_NB = 16
_BLK = 4096
_GRID = _TOTAL // _BLK


def _fused_body(x_ref, w_ref, benc_ref, starts_ref, ends_ref, out_ref):
    i = pl.program_id(0)
    h = jnp.maximum(
        jnp.dot(x_ref[...], w_ref[...], preferred_element_type=jnp.float32)
        + benc_ref[...], 0.0)

    rows = i * _BLK + jax.lax.broadcasted_iota(jnp.int32, (_NB, _BLK), 1)
    mask = ((rows >= starts_ref[...]) & (rows < ends_ref[...])
            ).astype(jnp.float32)
    out_ref[...] = jnp.dot(mask, h, preferred_element_type=jnp.float32
                           ).reshape(1, _NB, _D)


def kernel(x, bag_sizes, W_enc, b_enc, W_agg, b_agg):
    starts = bag_sizes[:_NB].reshape(_NB, 1)
    ends = bag_sizes[1:].reshape(_NB, 1)
    benc = b_enc.reshape(1, _D)

    partials = pl.pallas_call(
        _fused_body,
        grid=(_GRID,),
        in_specs=[
            pl.BlockSpec((_BLK, _D), lambda i: (i, 0)),
            pl.BlockSpec((_D, _D), lambda i: (0, 0)),
            pl.BlockSpec((1, _D), lambda i: (0, 0)),
            pl.BlockSpec((_NB, 1), lambda i: (0, 0)),
            pl.BlockSpec((_NB, 1), lambda i: (0, 0)),
        ],
        out_specs=pl.BlockSpec((1, _NB, _D), lambda i: (i, 0, 0)),
        out_shape=jax.ShapeDtypeStruct((_GRID, _NB, _D), jnp.float32),
        compiler_params=pltpu.CompilerParams(
            dimension_semantics=("parallel",)),
    )(x, W_enc, benc, starts, ends)

    sums = partials.sum(axis=0)
    counts = jnp.maximum((ends - starts).astype(jnp.float32), 1.0)
    means = sums / counts
    return means @ W_agg + b_agg
